# Initial kernel scaffold; baseline (speedup 1.0000x reference)
import jax
import jax.numpy as jnp
from jax.experimental import pallas as pl


def kernel(tensor, int_to_feat_matrix, extra_embeddings):
    raise NotImplementedError("write your pallas kernel here")


# speedup vs baseline: 3.4247x; 3.4247x over previous
import functools

import jax
import jax.numpy as jnp
from jax import lax
from jax.experimental import pallas as pl
from jax.experimental.pallas import tpu as pltpu
from jax.experimental.pallas import tpu_sc as plsc

MAX_COUNT = 255
EMBED_DIM = 128
NUM_CORES = 2
NUM_SUBCORES = 16
NUM_WORKERS = NUM_CORES * NUM_SUBCORES
CHUNK = 128


@functools.cache
def _build(n_chunks: int):
    b_per_w = n_chunks * CHUNK
    total = NUM_WORKERS * b_per_w
    mesh = plsc.VectorSubcoreMesh(core_axis_name="c", subcore_axis_name="s")

    @functools.partial(
        pl.kernel,
        out_type=jax.ShapeDtypeStruct((total, EMBED_DIM), jnp.float32),
        mesh=mesh,
        scratch_types=[
            pltpu.VMEM((n_chunks, CHUNK), jnp.int32),
            pltpu.VMEM((CHUNK, EMBED_DIM), jnp.float32),
            pltpu.SemaphoreType.DMA,
        ],
    )
    def gather_kernel(table_hbm, idx_hbm, out_hbm, idx_v, rows_v, sem):
        wid = lax.axis_index("s") * NUM_CORES + lax.axis_index("c")
        base = wid * b_per_w
        pltpu.sync_copy(idx_hbm.at[wid], idx_v)

        def body(i, carry):
            pltpu.async_copy(table_hbm.at[idx_v.at[i]], rows_v, sem).wait()
            pltpu.sync_copy(rows_v, out_hbm.at[pl.ds(base + i * CHUNK, CHUNK)])
            return carry

        lax.fori_loop(0, n_chunks, body, 0)

    return gather_kernel


def kernel(tensor, int_to_feat_matrix, extra_embeddings):
    batch, fields = tensor.shape
    total = batch * fields
    table = jnp.concatenate([int_to_feat_matrix, extra_embeddings], axis=0)
    b_per_w = total // NUM_WORKERS
    idx = tensor.astype(jnp.int32).reshape(NUM_WORKERS, b_per_w // CHUNK, CHUNK)
    out = _build(b_per_w // CHUNK)(table, idx)
    return out.reshape(batch, fields * EMBED_DIM)


# speedup vs baseline: 3.4378x; 1.0038x over previous
import functools

import jax
import jax.numpy as jnp
from jax import lax
from jax.experimental import pallas as pl
from jax.experimental.pallas import tpu as pltpu
from jax.experimental.pallas import tpu_sc as plsc

MAX_COUNT = 255
EMBED_DIM = 128
NUM_CORES = 2
NUM_SUBCORES = 16
NUM_WORKERS = NUM_CORES * NUM_SUBCORES
CHUNK = 128
NBUF = 4


@functools.cache
def _build(n_chunks: int):
    b_per_w = n_chunks * CHUNK
    total = NUM_WORKERS * b_per_w
    n_rounds = n_chunks // NBUF
    mesh = plsc.VectorSubcoreMesh(core_axis_name="c", subcore_axis_name="s")

    @functools.partial(
        pl.kernel,
        out_type=jax.ShapeDtypeStruct((total, EMBED_DIM), jnp.float32),
        mesh=mesh,
        scratch_types=[
            pltpu.VMEM((n_chunks, CHUNK), jnp.int32),
            [pltpu.VMEM((CHUNK, EMBED_DIM), jnp.float32) for _ in range(NBUF)],
            [pltpu.SemaphoreType.DMA for _ in range(NBUF)],
            [pltpu.SemaphoreType.DMA for _ in range(NBUF)],
        ],
    )
    def gather_kernel(table_hbm, idx_hbm, out_hbm, idx_v, rows, semg, sems):
        wid = lax.axis_index("s") * NUM_CORES + lax.axis_index("c")
        base = wid * b_per_w
        pltpu.sync_copy(idx_hbm.at[wid], idx_v)

        def start_gather(b, chunk):
            pltpu.async_copy(table_hbm.at[idx_v.at[chunk]], rows[b], semg[b])

        def wait_gather(b):
            pltpu.make_async_copy(
                out_hbm.at[pl.ds(0, CHUNK)], rows[b], semg[b]
            ).wait()

        def start_store(b, chunk):
            pltpu.async_copy(
                rows[b], out_hbm.at[pl.ds(base + chunk * CHUNK, CHUNK)], sems[b]
            )

        def wait_store(b):
            pltpu.make_async_copy(
                rows[b], out_hbm.at[pl.ds(0, CHUNK)], sems[b]
            ).wait()

        for b in range(NBUF):
            start_gather(b, b)

        def round_body(j, carry):
            for b in range(NBUF):
                wait_gather(b)
                start_store(b, j * NBUF + b)
            for b in range(NBUF):
                wait_store(b)
                start_gather(b, (j + 1) * NBUF + b)
            return carry

        lax.fori_loop(0, n_rounds - 1, round_body, 0)

        for b in range(NBUF):
            wait_gather(b)
            start_store(b, (n_rounds - 1) * NBUF + b)
        for b in range(NBUF):
            wait_store(b)

    return gather_kernel


def kernel(tensor, int_to_feat_matrix, extra_embeddings):
    batch, fields = tensor.shape
    total = batch * fields
    table = jnp.concatenate([int_to_feat_matrix, extra_embeddings], axis=0)
    b_per_w = total // NUM_WORKERS
    idx = tensor.astype(jnp.int32).reshape(NUM_WORKERS, b_per_w // CHUNK, CHUNK)
    out = _build(b_per_w // CHUNK)(table, idx)
    return out.reshape(batch, fields * EMBED_DIM)
